# baseline (speedup 1.0000x reference)
import functools

import jax
import jax.numpy as jnp
from jax import lax
from jax.experimental import pallas as pl
from jax.experimental.pallas import tpu as pltpu
from jax.experimental.pallas import tpu_sc as plsc

_T = 32768
_DIM = 2048
_N_EXPERTS = 8
_GROUP_SIZE = 4
_ROUTE_SCALE = 2.5
_BLOCK = 2048

_N_SLICES = 2
_T_SLICE = _T // _N_SLICES

_NC = 2
_NS = 16
_NW = _NC * _NS
_CHUNK = _T_SLICE // _NW
_L = 16
_STEPS = _CHUNK // _L


def _scores_block(x_ref, w_ref, s_ref):
    x = x_ref[...]
    w = w_ref[...]
    s = jax.lax.dot_general(
        w, x, (((1,), (1,)), ((), ())), preferred_element_type=jnp.float32
    )
    s_ref[...] = jax.nn.sigmoid(s)


def _sc_gate(s_hbm, wout_hbm, iout_hbm, sv, wv, iv):
    wid = lax.axis_index("s") * _NC + lax.axis_index("c")
    base = wid * _CHUNK
    pltpu.sync_copy(s_hbm.at[:, pl.ds(base, _CHUNK)], sv)

    negf2 = jnp.full((_L,), -2.0, jnp.float32)
    zero_i = jnp.zeros((_L,), jnp.int32)
    four_i = jnp.full((_L,), _GROUP_SIZE, jnp.int32)

    def body(j, carry):
        sl = pl.ds(j * _L, _L)
        s = [sv[e, sl] for e in range(_N_EXPERTS)]
        g0 = jnp.maximum(jnp.maximum(s[0], s[1]), jnp.maximum(s[2], s[3]))
        g1 = jnp.maximum(jnp.maximum(s[4], s[5]), jnp.maximum(s[6], s[7]))
        chosen0 = g0 >= g1
        c = [
            jnp.where(chosen0, s[e], s[e + _GROUP_SIZE])
            for e in range(_GROUP_SIZE)
        ]

        best = c[0]
        bidx = zero_i
        sec = negf2
        sidx = zero_i
        for e in range(1, _GROUP_SIZE):
            ev = jnp.full((_L,), e, jnp.int32)
            new_best = c[e] > best
            new_sec = jnp.logical_and(c[e] <= best, c[e] > sec)
            sec = jnp.where(new_best, best, jnp.where(new_sec, c[e], sec))
            sidx = jnp.where(new_best, bidx, jnp.where(new_sec, ev, sidx))
            best = jnp.where(new_best, c[e], best)
            bidx = jnp.where(new_best, ev, bidx)

        goff = jnp.where(chosen0, zero_i, four_i)
        bidx = bidx + goff
        sidx = sidx + goff
        scale = _ROUTE_SCALE / (best + sec)
        wv[0, sl] = best * scale
        wv[1, sl] = sec * scale
        iv[0, sl] = bidx
        iv[1, sl] = sidx
        return carry

    lax.fori_loop(0, _STEPS, body, 0)
    pltpu.sync_copy(wv, wout_hbm.at[:, pl.ds(base, _CHUNK)])
    pltpu.sync_copy(iv, iout_hbm.at[:, pl.ds(base, _CHUNK)])


@jax.jit
def kernel(x, weight):
    blocks_per_slice = _T_SLICE // _BLOCK

    gate = functools.partial(
        pl.kernel,
        mesh=plsc.VectorSubcoreMesh(core_axis_name="c", subcore_axis_name="s"),
        out_type=[
            jax.ShapeDtypeStruct((2, _T_SLICE), jnp.float32),
            jax.ShapeDtypeStruct((2, _T_SLICE), jnp.int32),
        ],
        scratch_types=[
            pltpu.VMEM((_N_EXPERTS, _CHUNK), jnp.float32),
            pltpu.VMEM((2, _CHUNK), jnp.float32),
            pltpu.VMEM((2, _CHUNK), jnp.int32),
        ],
    )(_sc_gate)

    wts, its = [], []
    for h in range(_N_SLICES):
        scores_t = pl.pallas_call(
            _scores_block,
            grid=(blocks_per_slice,),
            in_specs=[
                pl.BlockSpec(
                    (_BLOCK, _DIM),
                    lambda i, h=h: (i + h * blocks_per_slice, 0),
                ),
                pl.BlockSpec((_N_EXPERTS, _DIM), lambda i: (0, 0)),
            ],
            out_specs=pl.BlockSpec((_N_EXPERTS, _BLOCK), lambda i: (0, i)),
            out_shape=jax.ShapeDtypeStruct((_N_EXPERTS, _T_SLICE), jnp.float32),
        )(x, weight)
        wt, it = gate(scores_t)
        wts.append(wt)
        its.append(it)

    wt = jnp.concatenate(wts, axis=1)
    it = jnp.concatenate(its, axis=1)
    return wt.T, it.T


# speedup vs baseline: 1.3295x; 1.3295x over previous
import jax
import jax.numpy as jnp
from jax.experimental import pallas as pl

_T = 32768
_DIM = 2048
_N_EXPERTS = 8
_GROUP_SIZE = 4
_ROUTE_SCALE = 2.5
_BLOCK = 2048


def _gate_block(x_ref, w_ref, wout_ref, iout_ref):
    x = x_ref[...]
    w = w_ref[...]
    s = jax.lax.dot_general(
        w, x, (((1,), (1,)), ((), ())), preferred_element_type=jnp.float32
    )
    s = jax.nn.sigmoid(s)

    g0 = jnp.max(s[0:4], axis=0, keepdims=True)
    g1 = jnp.max(s[4:8], axis=0, keepdims=True)
    chosen0 = g0 >= g1
    c = jnp.where(chosen0, s[0:4], s[4:8])

    best = c[0:1]
    bidx = jnp.zeros_like(best)
    sec = jnp.full_like(best, -2.0)
    sidx = jnp.zeros_like(best)
    for e in range(1, _GROUP_SIZE):
        ce = c[e : e + 1]
        ev = jnp.full_like(best, float(e))
        new_best = ce > best
        new_sec = jnp.logical_and(ce <= best, ce > sec)
        sec = jnp.where(new_best, best, jnp.where(new_sec, ce, sec))
        sidx = jnp.where(new_best, bidx, jnp.where(new_sec, ev, sidx))
        best = jnp.where(new_best, ce, best)
        bidx = jnp.where(new_best, ev, bidx)

    goff = jnp.where(chosen0, 0.0, float(_GROUP_SIZE))
    scale = _ROUTE_SCALE / (best + sec)
    wout_ref[0:1, :] = best * scale
    wout_ref[1:2, :] = sec * scale
    iout_ref[0:1, :] = (bidx + goff).astype(jnp.int32)
    iout_ref[1:2, :] = (sidx + goff).astype(jnp.int32)


@jax.jit
def kernel(x, weight):
    n_blocks = _T // _BLOCK
    wt, it = pl.pallas_call(
        _gate_block,
        grid=(n_blocks,),
        in_specs=[
            pl.BlockSpec((_BLOCK, _DIM), lambda i: (i, 0)),
            pl.BlockSpec((_N_EXPERTS, _DIM), lambda i: (0, 0)),
        ],
        out_specs=[
            pl.BlockSpec((2, _BLOCK), lambda i: (0, i)),
            pl.BlockSpec((2, _BLOCK), lambda i: (0, i)),
        ],
        out_shape=[
            jax.ShapeDtypeStruct((2, _T), jnp.float32),
            jax.ShapeDtypeStruct((2, _T), jnp.int32),
        ],
    )(x, weight)
    return wt.T, it.T
